# baseline (speedup 1.0000x reference)
import jax
import jax.numpy as jnp
from jax import lax
from jax.experimental import pallas as pl
from jax.experimental.pallas import tpu as pltpu
from jax.experimental.pallas import tpu_sc as plsc

N = 10000
D = 128
B = 1024
M = 64
H = 32
NPAD = 10008

NC, NS = 2, 16
NW = NC * NS
BM = B * M
BP = B * 2
NB_W = BM // NW
NCHUNK = 8
CROWS = NB_W // NCHUNK
PB_W = BP // NW

QB = 128
RB = QB * M


def _gather_rows(table, nidx_hbm, out_n, wid, idxvs, nbufs, semgs, semws):
    rows_w = out_n.shape[0] // NW
    nchunk = rows_w // CROWS
    cpg = [None, None]
    cpw = [None, None]
    pltpu.sync_copy(nidx_hbm.at[pl.ds(wid * rows_w, CROWS)], idxvs[0])
    cpg[0] = pltpu.async_copy(table.at[idxvs[0]], nbufs[0], semgs[0])
    for c in range(nchunk):
        cur = c % 2
        nxt = (c + 1) % 2
        if c + 1 < nchunk:
            pltpu.sync_copy(
                nidx_hbm.at[pl.ds(wid * rows_w + (c + 1) * CROWS, CROWS)],
                idxvs[nxt])
            if cpw[nxt] is not None:
                cpw[nxt].wait()
            cpg[nxt] = pltpu.async_copy(
                table.at[idxvs[nxt]], nbufs[nxt], semgs[nxt])
        cpg[cur].wait()
        cpw[cur] = pltpu.async_copy(
            nbufs[cur], out_n.at[pl.ds(wid * rows_w + c * CROWS, CROWS)],
            semws[cur])
    cpw[0].wait()
    cpw[1].wait()


def _sc_gather_a(t2_hbm, emb_hbm, nidx_hbm, didx_hbm, out_n, out_p,
                 idxn_v0, idxn_v1, idxp_v, nbuf0, nbuf1, pair_v,
                 semg0, semg1, semw0, semw1, semp):
    wid = lax.axis_index("s") * NC + lax.axis_index("c")
    pltpu.sync_copy(didx_hbm.at[pl.ds(wid * PB_W, PB_W)], idxp_v)
    cpp = pltpu.async_copy(emb_hbm.at[idxp_v], pair_v, semp)
    _gather_rows(t2_hbm, nidx_hbm, out_n, wid, (idxn_v0, idxn_v1),
                 (nbuf0, nbuf1), (semg0, semg1), (semw0, semw1))
    cpp.wait()
    pltpu.sync_copy(pair_v, out_p.at[pl.ds(wid * PB_W, PB_W)])


def _sc_gather_b(t2_hbm, nidx_hbm, out_n,
                 idxn_v0, idxn_v1, nbuf0, nbuf1,
                 semg0, semg1, semw0, semw1):
    wid = lax.axis_index("s") * NC + lax.axis_index("c")
    _gather_rows(t2_hbm, nidx_hbm, out_n, wid, (idxn_v0, idxn_v1),
                 (nbuf0, nbuf1), (semg0, semg1), (semw0, semw1))


def _table_body(emb_ref, dw_ref, w1b_ref, t2_ref):
    w1b = w1b_ref[...]
    t2_ref[pl.ds(0, N), :] = jnp.dot(
        emb_ref[...].astype(jnp.bfloat16), w1b,
        preferred_element_type=jnp.float32)
    dwb = jnp.broadcast_to(dw_ref[...], (8, D)).astype(jnp.bfloat16)
    t2_ref[pl.ds(N, 8), :] = jnp.dot(
        dwb, w1b, preferred_element_type=jnp.float32)


def _mlp_body(pair_ref, t2_ref, w1t_ref, b1_ref, wa_ref,
              ba_ref, g_ref, beta_ref, wb_ref, lg_ref):
    pr = pair_ref[...].reshape(QB, 2, D)
    kq = (pr[:, 0, :] + pr[:, 1, :]) * 0.5
    e1b = kq.astype(jnp.bfloat16)
    a1 = jnp.dot(e1b, w1t_ref[...], preferred_element_type=jnp.float32)
    a1r = jnp.broadcast_to(a1[:, None, :], (QB, M, D)).reshape(RB, D)
    kqr = jnp.broadcast_to(kq[:, None, :], (QB, M, D)).reshape(RB, D)
    c1 = a1r + t2_ref[...]
    e = kqr + (c1 + b1_ref[...])
    ebf = e.astype(jnp.bfloat16)
    c2 = jnp.dot(ebf, wa_ref[...], preferred_element_type=jnp.float32)
    h = c2 + ba_ref[...]
    mu = jnp.mean(h, axis=-1, keepdims=True)
    var = jnp.mean((h - mu) ** 2, axis=-1, keepdims=True)
    hn = (h - mu) / jnp.sqrt(var + 1e-5) * g_ref[...] + beta_ref[...]
    t = jnp.tanh(hn)
    c3 = jnp.dot(t, wb_ref[...], preferred_element_type=jnp.float32)
    lg_ref[...] = c3.reshape(QB, M)


def _sample_body(lg_ref, bb_ref, gum_ref, nidx_ref, probs_ref, samp_ref):
    l = lg_ref[...] + bb_ref[0, 0]
    mx = jnp.max(l, axis=-1, keepdims=True)
    ex = jnp.exp(l - mx)
    s = jnp.sum(ex, axis=-1, keepdims=True)
    probs = ex / s
    probs_ref[...] = probs
    z = gum_ref[...] + jnp.log(probs + 1e-20)
    zmax = jnp.max(z, axis=-1, keepdims=True)
    iota = lax.broadcasted_iota(jnp.int32, (B, M), 1)
    samp = jnp.min(jnp.where(z == zmax, iota, M), axis=-1)
    samp_ref[...] = jnp.sum(
        jnp.where(iota == samp[:, None], nidx_ref[...], 0),
        axis=-1, keepdims=True)


def kernel(data_idx, adj_matrix, edge_rel, embeddings, neighbor_idx, done_w,
           W1, b1, Wa, ba, g, beta, Wb, bb):
    del adj_matrix, edge_rel
    f32 = jnp.float32

    t2_tab = pl.pallas_call(
        _table_body,
        out_shape=jax.ShapeDtypeStruct((NPAD, D), f32),
    )(embeddings, done_w, W1[D:, :])

    nidx_flat = neighbor_idx.reshape(BM).astype(jnp.int32)
    didx_flat = data_idx.reshape(BP).astype(jnp.int32)
    BH = BM // 2

    mesh = plsc.VectorSubcoreMesh(core_axis_name="c", subcore_axis_name="s")
    sc_params = pltpu.CompilerParams(use_tc_tiling_on_sc=False)
    nbuf_scratch = [
        pltpu.VMEM((CROWS,), jnp.int32),
        pltpu.VMEM((CROWS,), jnp.int32),
        pltpu.VMEM((CROWS, D), f32),
        pltpu.VMEM((CROWS, D), f32),
        pltpu.SemaphoreType.DMA,
        pltpu.SemaphoreType.DMA,
        pltpu.SemaphoreType.DMA,
        pltpu.SemaphoreType.DMA,
    ]
    NSPLIT = 4
    BQ = BM // NSPLIT
    gathered = []
    gathered_p = None
    for k in range(NSPLIT):
        nidx_k = nidx_flat[k * BQ:(k + 1) * BQ]
        if k == 0:
            g_k, gathered_p = pl.kernel(
                _sc_gather_a,
                mesh=mesh,
                compiler_params=sc_params,
                out_type=[jax.ShapeDtypeStruct((BQ, D), f32),
                          jax.ShapeDtypeStruct((BP, D), f32)],
                scratch_types=nbuf_scratch[:2]
                + [pltpu.VMEM((PB_W,), jnp.int32)]
                + nbuf_scratch[2:4] + [pltpu.VMEM((PB_W, D), f32)]
                + nbuf_scratch[4:] + [pltpu.SemaphoreType.DMA],
            )(t2_tab, embeddings, nidx_k, didx_flat)
        else:
            g_k = pl.kernel(
                _sc_gather_b,
                mesh=mesh,
                compiler_params=sc_params,
                out_type=jax.ShapeDtypeStruct((BQ, D), f32),
                scratch_types=nbuf_scratch,
            )(t2_tab, nidx_k)
        gathered.append(g_k)

    nq = B // NSPLIT // QB
    mlp_specs = dict(
        grid=(nq,),
        in_specs=[
            pl.BlockSpec((2 * QB, D), lambda i: (i, 0)),
            pl.BlockSpec((RB, D), lambda i: (i, 0)),
            pl.BlockSpec((D, D), lambda i: (0, 0)),
            pl.BlockSpec((1, D), lambda i: (0, 0)),
            pl.BlockSpec((D, H), lambda i: (0, 0)),
            pl.BlockSpec((1, H), lambda i: (0, 0)),
            pl.BlockSpec((1, H), lambda i: (0, 0)),
            pl.BlockSpec((1, H), lambda i: (0, 0)),
            pl.BlockSpec((H, 1), lambda i: (0, 0)),
        ],
        out_specs=pl.BlockSpec((QB, M), lambda i: (i, 0)),
        out_shape=jax.ShapeDtypeStruct((B // NSPLIT, M), f32),
    )
    wargs = (W1[:D, :], b1.reshape(1, D), Wa, ba.reshape(1, H),
             g.reshape(1, H), beta.reshape(1, H), Wb)
    PQ = BP // NSPLIT
    logits_parts = [
        pl.pallas_call(_mlp_body, **mlp_specs)(
            gathered_p[k * PQ:(k + 1) * PQ], gathered[k], *wargs)
        for k in range(NSPLIT)
    ]
    logits = jnp.concatenate(logits_parts, axis=0)

    gum = jax.random.gumbel(jax.random.key(42), (B, M), f32)

    probs, sampled = pl.pallas_call(
        _sample_body,
        out_shape=[jax.ShapeDtypeStruct((B, M), f32),
                   jax.ShapeDtypeStruct((B, 1), jnp.int32)],
    )(logits, bb.reshape(1, 1), gum, neighbor_idx.astype(jnp.int32))

    return (probs, sampled.reshape(B))


# speedup vs baseline: 1.0494x; 1.0494x over previous
import jax
import jax.numpy as jnp
from jax import lax
from jax.experimental import pallas as pl
from jax.experimental.pallas import tpu as pltpu
from jax.experimental.pallas import tpu_sc as plsc

N = 10000
D = 128
B = 1024
M = 64
H = 32
NPAD = 10008

NC, NS = 2, 16
NW = NC * NS
BM = B * M
BP = B * 2
NB_W = BM // NW
NCHUNK = 8
CROWS = NB_W // NCHUNK
PB_W = BP // NW

QB = 128
RB = QB * M


def _gather_rows(table, nidx_hbm, out_n, wid, idxvs, nbufs, semgs, semws):
    rows_w = out_n.shape[0] // NW
    nchunk = rows_w // CROWS
    cpg = [None, None]
    cpw = [None, None]
    pltpu.sync_copy(nidx_hbm.at[pl.ds(wid * rows_w, CROWS)], idxvs[0])
    cpg[0] = pltpu.async_copy(table.at[idxvs[0]], nbufs[0], semgs[0])
    for c in range(nchunk):
        cur = c % 2
        nxt = (c + 1) % 2
        if c + 1 < nchunk:
            pltpu.sync_copy(
                nidx_hbm.at[pl.ds(wid * rows_w + (c + 1) * CROWS, CROWS)],
                idxvs[nxt])
            if cpw[nxt] is not None:
                cpw[nxt].wait()
            cpg[nxt] = pltpu.async_copy(
                table.at[idxvs[nxt]], nbufs[nxt], semgs[nxt])
        cpg[cur].wait()
        cpw[cur] = pltpu.async_copy(
            nbufs[cur], out_n.at[pl.ds(wid * rows_w + c * CROWS, CROWS)],
            semws[cur])
    cpw[0].wait()
    cpw[1].wait()


def _sc_gather_a(t2_hbm, emb_hbm, nidx_hbm, didx_hbm, out_n, out_p,
                 idxn_v0, idxn_v1, idxp_v, nbuf0, nbuf1, pair_v,
                 semg0, semg1, semw0, semw1, semp):
    wid = lax.axis_index("s") * NC + lax.axis_index("c")
    pltpu.sync_copy(didx_hbm.at[pl.ds(wid * PB_W, PB_W)], idxp_v)
    cpp = pltpu.async_copy(emb_hbm.at[idxp_v], pair_v, semp)
    _gather_rows(t2_hbm, nidx_hbm, out_n, wid, (idxn_v0, idxn_v1),
                 (nbuf0, nbuf1), (semg0, semg1), (semw0, semw1))
    cpp.wait()
    pltpu.sync_copy(pair_v, out_p.at[pl.ds(wid * PB_W, PB_W)])


def _sc_gather_b(t2_hbm, nidx_hbm, out_n,
                 idxn_v0, idxn_v1, nbuf0, nbuf1,
                 semg0, semg1, semw0, semw1):
    wid = lax.axis_index("s") * NC + lax.axis_index("c")
    _gather_rows(t2_hbm, nidx_hbm, out_n, wid, (idxn_v0, idxn_v1),
                 (nbuf0, nbuf1), (semg0, semg1), (semw0, semw1))


def _table_body(emb_ref, dw_ref, w1b_ref, t2_ref):
    w1b = w1b_ref[...]
    t2_ref[pl.ds(0, N), :] = jnp.dot(
        emb_ref[...].astype(jnp.bfloat16), w1b,
        preferred_element_type=jnp.float32)
    dwb = jnp.broadcast_to(dw_ref[...], (8, D)).astype(jnp.bfloat16)
    t2_ref[pl.ds(N, 8), :] = jnp.dot(
        dwb, w1b, preferred_element_type=jnp.float32)


def _mlp_body(pair_ref, t2_ref, w1t_ref, b1_ref, wa_ref,
              ba_ref, g_ref, beta_ref, wb_ref, lg_ref):
    pr = pair_ref[...].reshape(QB, 2, D)
    kq = (pr[:, 0, :] + pr[:, 1, :]) * 0.5
    e1b = kq.astype(jnp.bfloat16)
    a1 = jnp.dot(e1b, w1t_ref[...], preferred_element_type=jnp.float32)
    a1r = jnp.broadcast_to(a1[:, None, :], (QB, M, D)).reshape(RB, D)
    kqr = jnp.broadcast_to(kq[:, None, :], (QB, M, D)).reshape(RB, D)
    c1 = a1r + t2_ref[...]
    e = kqr + (c1 + b1_ref[...])
    ebf = e.astype(jnp.bfloat16)
    c2 = jnp.dot(ebf, wa_ref[...], preferred_element_type=jnp.float32)
    h = c2 + ba_ref[...]
    mu = jnp.mean(h, axis=-1, keepdims=True)
    var = jnp.mean((h - mu) ** 2, axis=-1, keepdims=True)
    hn = (h - mu) / jnp.sqrt(var + 1e-5) * g_ref[...] + beta_ref[...]
    t = jnp.tanh(hn)
    c3 = jnp.dot(t, wb_ref[...], preferred_element_type=jnp.float32)
    lg_ref[...] = c3.reshape(QB, M)


def _sample_body(lg_ref, bb_ref, gum_ref, nidx_ref, probs_ref, samp_ref):
    l = lg_ref[...] + bb_ref[0, 0]
    mx = jnp.max(l, axis=-1, keepdims=True)
    ex = jnp.exp(l - mx)
    s = jnp.sum(ex, axis=-1, keepdims=True)
    probs = ex / s
    probs_ref[...] = probs
    z = gum_ref[...] + jnp.log(probs + 1e-20)
    zmax = jnp.max(z, axis=-1, keepdims=True)
    iota = lax.broadcasted_iota(jnp.int32, (B, M), 1)
    samp = jnp.min(jnp.where(z == zmax, iota, M), axis=-1)
    samp_ref[...] = jnp.sum(
        jnp.where(iota == samp[:, None], nidx_ref[...], 0),
        axis=-1, keepdims=True)


def kernel(data_idx, adj_matrix, edge_rel, embeddings, neighbor_idx, done_w,
           W1, b1, Wa, ba, g, beta, Wb, bb):
    del adj_matrix, edge_rel
    f32 = jnp.float32

    t2_tab = pl.pallas_call(
        _table_body,
        out_shape=jax.ShapeDtypeStruct((NPAD, D), f32),
    )(embeddings, done_w, W1[D:, :])

    nidx_flat = neighbor_idx.reshape(BM).astype(jnp.int32)
    didx_flat = data_idx.reshape(BP).astype(jnp.int32)
    BH = BM // 2

    mesh = plsc.VectorSubcoreMesh(core_axis_name="c", subcore_axis_name="s")
    sc_params = pltpu.CompilerParams(use_tc_tiling_on_sc=False)
    nbuf_scratch = [
        pltpu.VMEM((CROWS,), jnp.int32),
        pltpu.VMEM((CROWS,), jnp.int32),
        pltpu.VMEM((CROWS, D), f32),
        pltpu.VMEM((CROWS, D), f32),
        pltpu.SemaphoreType.DMA,
        pltpu.SemaphoreType.DMA,
        pltpu.SemaphoreType.DMA,
        pltpu.SemaphoreType.DMA,
    ]
    NSPLIT = 2
    BQ = BM // NSPLIT
    gathered = []
    gathered_p = None
    for k in range(NSPLIT):
        nidx_k = nidx_flat[k * BQ:(k + 1) * BQ]
        if k == 0:
            g_k, gathered_p = pl.kernel(
                _sc_gather_a,
                mesh=mesh,
                compiler_params=sc_params,
                out_type=[jax.ShapeDtypeStruct((BQ, D), f32),
                          jax.ShapeDtypeStruct((BP, D), f32)],
                scratch_types=nbuf_scratch[:2]
                + [pltpu.VMEM((PB_W,), jnp.int32)]
                + nbuf_scratch[2:4] + [pltpu.VMEM((PB_W, D), f32)]
                + nbuf_scratch[4:] + [pltpu.SemaphoreType.DMA],
            )(t2_tab, embeddings, nidx_k, didx_flat)
        else:
            g_k = pl.kernel(
                _sc_gather_b,
                mesh=mesh,
                compiler_params=sc_params,
                out_type=jax.ShapeDtypeStruct((BQ, D), f32),
                scratch_types=nbuf_scratch,
            )(t2_tab, nidx_k)
        gathered.append(g_k)

    nq = B // NSPLIT // QB
    mlp_specs = dict(
        grid=(nq,),
        in_specs=[
            pl.BlockSpec((2 * QB, D), lambda i: (i, 0)),
            pl.BlockSpec((RB, D), lambda i: (i, 0)),
            pl.BlockSpec((D, D), lambda i: (0, 0)),
            pl.BlockSpec((1, D), lambda i: (0, 0)),
            pl.BlockSpec((D, H), lambda i: (0, 0)),
            pl.BlockSpec((1, H), lambda i: (0, 0)),
            pl.BlockSpec((1, H), lambda i: (0, 0)),
            pl.BlockSpec((1, H), lambda i: (0, 0)),
            pl.BlockSpec((H, 1), lambda i: (0, 0)),
        ],
        out_specs=pl.BlockSpec((QB, M), lambda i: (i, 0)),
        out_shape=jax.ShapeDtypeStruct((B // NSPLIT, M), f32),
    )
    wargs = (W1[:D, :], b1.reshape(1, D), Wa, ba.reshape(1, H),
             g.reshape(1, H), beta.reshape(1, H), Wb)
    PQ = BP // NSPLIT
    logits_parts = [
        pl.pallas_call(_mlp_body, **mlp_specs)(
            gathered_p[k * PQ:(k + 1) * PQ], gathered[k], *wargs)
        for k in range(NSPLIT)
    ]
    logits = jnp.concatenate(logits_parts, axis=0)

    gum = jax.random.gumbel(jax.random.key(42), (B, M), f32)

    probs, sampled = pl.pallas_call(
        _sample_body,
        out_shape=[jax.ShapeDtypeStruct((B, M), f32),
                   jax.ShapeDtypeStruct((B, 1), jnp.int32)],
    )(logits, bb.reshape(1, 1), gum, neighbor_idx.astype(jnp.int32))

    return (probs, sampled.reshape(B))
